# baseline (speedup 1.0000x reference)
import functools

import jax
import jax.numpy as jnp
from jax import lax
from jax.experimental import pallas as pl
from jax.experimental.pallas import tpu as pltpu
from jax.experimental.pallas import tpu_sc as plsc

_RB = 256
_CAP = 2048
_SLACK = 0
_NSLOT = 20480
_SENTINEL = 0x7FFFFFF0
_BISECT_PHASE2 = True
_BISECT_OVERFLOW = True


@functools.cache
def _build_collect(B: int, V: int, D: int):
    info = plsc.get_sparse_core_info()
    L = info.num_lanes
    nc = info.num_cores
    nw = nc * info.num_subcores
    nblk = (V + _RB - 1) // _RB
    tmax = (nblk + nw - 1) // nw
    n_chunks = 4
    chunk = B // n_chunks

    mesh = plsc.VectorSubcoreMesh(core_axis_name="c", subcore_axis_name="s")

    @functools.partial(
        pl.kernel,
        mesh=mesh,
        out_type=[
            jax.ShapeDtypeStruct((_NSLOT, D), jnp.float32),
            jax.ShapeDtypeStruct((_NSLOT,), jnp.int32),
        ],
        scratch_types=[
            pltpu.VMEM((chunk,), jnp.int32),
            pltpu.VMEM((_CAP + L,), jnp.int32),
            pltpu.VMEM((_CAP + L,), jnp.int32),
            pltpu.VMEM((2, _RB, D), jnp.float32),
            pltpu.VMEM((128, D), jnp.float32),
            pltpu.VMEM((128,), jnp.int32),
            pltpu.VMEM((2 * L,), jnp.int32),
            pltpu.VMEM((D,), jnp.float32),
            pltpu.VMEM((nw, L), jnp.int32),
            pltpu.SemaphoreType.DMA,
            pltpu.SemaphoreType.DMA,
        ],
        compiler_params=pltpu.CompilerParams(needs_layout_passes=False),
    )
    def collect_kernel(
        idx_hbm,
        table_hbm,
        base_hbm,
        rows_hbm,
        pos_hbm,
        dchunk,
        idxl,
        jl,
        buf,
        ext,
        pbuf,
        jtmp,
        rtmp,
        basev,
        sem0,
        sem1,
    ):
        w = lax.axis_index("s") * nc + lax.axis_index("c")
        lanes = lax.iota(jnp.int32, L)
        neg1 = jnp.full((L,), -1, jnp.int32)

        pltpu.sync_copy(base_hbm, basev)
        bvec = basev[w, pl.ds(0, L)]
        bw = pl.multiple_of(bvec[0], 128)
        bw1 = pl.multiple_of(bvec[1], 128)

        def reset_pbuf():
            for i in range(128 // L):
                pbuf[pl.ds(i * L, L)] = neg1

        reset_pbuf()
        fill_end = jnp.where(w == nw - 1, jnp.int32(_NSLOT), bw1)
        nfill = (fill_end - bw) // 128

        def fill_body(f, carry):
            pltpu.sync_copy(pbuf, pos_hbm.at[pl.ds(bw + f * 128, 128)])
            return carry

        lax.fori_loop(0, nfill, fill_body, 0)

        def emit_append(jscalar, ec, fl):
            plsc.store_scatter(
                pbuf, [jnp.full((L,), ec, jnp.int32)],
                jnp.full((L,), jscalar, jnp.int32), mask=lanes == 0,
            )
            ec = ec + 1

            def flush(fl):
                pltpu.sync_copy(ext, rows_hbm.at[pl.ds(bw + fl * 128, 128)])
                pltpu.sync_copy(pbuf, pos_hbm.at[pl.ds(bw + fl * 128, 128)])
                reset_pbuf()
                return fl + 1

            fl = lax.cond(ec == 128, flush, lambda f: f, fl)
            ec = jnp.where(ec == 128, 0, ec)
            return ec, fl

        def scan_chunk(c, carry):
            pltpu.sync_copy(idx_hbm.at[pl.ds(c * chunk, chunk)], dchunk)

            def scan_vec(g, carry):
                cnt, ec, fl = carry
                v = dchunk[pl.ds(g * L, L)]
                mine = lax.bitwise_and(lax.shift_right_logical(v, 8), nw - 1) == w
                k = plsc.all_reduce_population_count(mine)[0]
                jv = lanes + (c * chunk + g * L)
                in_cap = cnt + k <= _CAP

                @pl.when(jnp.logical_and(k > 0, in_cap))
                def _listed():
                    pv = cnt + plsc.cumsum(mine.astype(jnp.int32)) - 1
                    plsc.store_scatter(idxl, [pv], v, mask=mine)
                    plsc.store_scatter(jl, [pv], jv, mask=mine)

                def _overflow(carry):
                    ec, fl = carry
                    mine_i32 = mine.astype(jnp.int32)
                    for i in range(L):
                        def do_ov(carry, i=i):
                            ec, fl = carry
                            pltpu.sync_copy(table_hbm.at[v[i]], rtmp)
                            for kk in range(D // L):
                                ext[ec, pl.ds(kk * L, L)] = rtmp[pl.ds(kk * L, L)]
                            return emit_append(jv[i], ec, fl)

                        carry = lax.cond(mine_i32[i] != 0, do_ov, lambda cc: cc, (ec, fl))
                        ec, fl = carry
                    return ec, fl

                if _BISECT_OVERFLOW:
                    ec, fl = lax.cond(
                        jnp.logical_and(k > 0, jnp.logical_not(in_cap)),
                        _overflow,
                        lambda cc: cc,
                        (ec, fl),
                    )
                cnt = jnp.where(in_cap, cnt + k, cnt)
                return cnt, ec, fl

            return lax.fori_loop(0, chunk // L, scan_vec, carry)

        lc, ec, fl = lax.fori_loop(
            0, n_chunks, scan_chunk, (jnp.int32(0), jnp.int32(0), jnp.int32(0))
        )

        plsc.store_scatter(
            idxl, [lc + lanes], jnp.full((L,), _SENTINEL, jnp.int32)
        )
        nlv = (lc + L - 1) // L

        def blk_start(t):
            b = w + nw * t
            return jnp.minimum(b * _RB, V - _RB)

        def blk_valid(t):
            return w + nw * t < nblk

        def issue(t, p_static):
            @pl.when(blk_valid(t))
            def _():
                sem = sem0 if p_static == 0 else sem1
                pltpu.async_copy(
                    table_hbm.at[pl.ds(blk_start(t), _RB)], buf.at[p_static], sem
                )

        issue(jnp.int32(0), 0)
        issue(jnp.int32(1), 1)

        def block_body(t, carry):
            def with_parity(p_static, carry):
                sem = sem0 if p_static == 0 else sem1

                @pl.when(blk_valid(t))
                def _wait():
                    pltpu.make_async_copy(
                        table_hbm.at[pl.ds(0, _RB)], buf.at[p_static], sem
                    ).wait()

                start = blk_start(t)

                def scan_list(lv, carry):
                    ec, fl = carry
                    lvv = idxl[pl.ds(lv * L, L)]
                    m = lax.shift_right_logical(lvv, 13) == t
                    km = plsc.all_reduce_population_count(m)[0]

                    def matched(carry):
                        ec, fl = carry
                        pv = plsc.cumsum(m.astype(jnp.int32)) - 1
                        plsc.store_scatter(jtmp, [pv], lvv, mask=m)
                        jlv = jl[pl.ds(lv * L, L)]
                        plsc.store_scatter(jtmp, [pv + L], jlv, mask=m)

                        def entry(e, carry):
                            ec, fl = carry
                            espl = jnp.full((L,), e, jnp.int32)
                            ev0 = plsc.load_gather(jtmp, [espl])[0]
                            jev = plsc.load_gather(jtmp, [espl + L])[0]
                            l = ev0 - start
                            for kk in range(D // L):
                                ext[ec, pl.ds(kk * L, L)] = buf[
                                    p_static, l, pl.ds(kk * L, L)
                                ]
                            return emit_append(jev, ec, fl)

                        return lax.fori_loop(0, km, entry, (ec, fl))

                    return lax.cond(km > 0, matched, lambda cc: cc, (ec, fl))

                carry = lax.fori_loop(0, nlv, scan_list, carry)
                issue(t + 2, p_static)
                return carry

            return lax.cond(
                lax.rem(t, 2) == 0,
                lambda cc: with_parity(0, cc),
                lambda cc: with_parity(1, cc),
                carry,
            )

        if _BISECT_PHASE2:
            ec, fl = lax.fori_loop(0, tmax, block_body, (ec, fl))

        @pl.when(ec > 0)
        def _final_flush():
            pltpu.sync_copy(ext, rows_hbm.at[pl.ds(bw + fl * 128, 128)])
            pltpu.sync_copy(pbuf, pos_hbm.at[pl.ds(bw + fl * 128, 128)])

    return collect_kernel


@functools.cache
def _build_unscatter(B: int, D: int):
    info = plsc.get_sparse_core_info()
    nc = info.num_cores
    nw = nc * info.num_subcores
    spw = _NSLOT // nw
    npv = spw // 128

    mesh = plsc.VectorSubcoreMesh(core_axis_name="c", subcore_axis_name="s")

    @functools.partial(
        pl.kernel,
        mesh=mesh,
        out_type=jax.ShapeDtypeStruct((B, D), jnp.float32),
        scratch_types=[
            pltpu.VMEM((npv, 128), jnp.int32),
            pltpu.VMEM((spw, D), jnp.float32),
            pltpu.SemaphoreType.DMA,
        ],
        compiler_params=pltpu.CompilerParams(use_tc_tiling_on_sc=False),
    )
    def unscatter_kernel(rows_hbm, pos_hbm, out_hbm, pv, rv, sem):
        w = lax.axis_index("s") * nc + lax.axis_index("c")
        base = w * spw
        pltpu.sync_copy(pos_hbm.at[pl.ds(w * npv, npv)], pv)
        pltpu.sync_copy(rows_hbm.at[pl.ds(base, spw)], rv)
        copies = []
        for k in range(npv):
            copies.append(
                pltpu.async_copy(
                    rv.at[pl.ds(k * 128, 128)],
                    out_hbm.at[plsc.Indices(pv.at[k], ignored_value=-1)],
                    sem,
                )
            )
        for c in copies:
            c.wait()

    return unscatter_kernel


def kernel(data, emb):
    (B,) = data.shape
    V, D = emb.shape
    nw = 32
    tile_of = lax.bitwise_and(lax.shift_right_logical(data, 8), nw - 1)
    counts = jnp.zeros((nw,), jnp.int32).at[tile_of].add(1)
    regsz = ((counts + 127) // 128) * 128 + _SLACK
    base = jnp.concatenate(
        [jnp.zeros((1,), jnp.int32), jnp.cumsum(regsz, dtype=jnp.int32)]
    )
    base_pairs = jnp.pad(
        jnp.stack([base[:nw], base[1 : nw + 1]], axis=1), ((0, 0), (0, 14))
    )
    rows, pos = _build_collect(B, V, D)(data, emb, base_pairs)
    pos2 = pos.reshape(_NSLOT // 128, 128)
    return _build_unscatter(B, D)(rows, pos2)


# speedup vs baseline: 2.4255x; 2.4255x over previous
import functools

import jax
import jax.numpy as jnp
from jax import lax
from jax.experimental import pallas as pl
from jax.experimental.pallas import tpu as pltpu
from jax.experimental.pallas import tpu_sc as plsc

_RB = 256
_CAP = 2048
_SLACK = 0
_NSLOT = 20480
_SENTINEL = 0x7FFFFFF0
_BISECT_PHASE2 = True
_BISECT_OVERFLOW = True


@functools.cache
def _build_collect(B: int, V: int, D: int):
    info = plsc.get_sparse_core_info()
    L = info.num_lanes
    nc = info.num_cores
    nw = nc * info.num_subcores
    nblk = (V + _RB - 1) // _RB
    tmax = (nblk + nw - 1) // nw
    n_chunks = 4
    chunk = B // n_chunks

    mesh = plsc.VectorSubcoreMesh(core_axis_name="c", subcore_axis_name="s")

    @functools.partial(
        pl.kernel,
        mesh=mesh,
        out_type=[
            jax.ShapeDtypeStruct((_NSLOT, D), jnp.float32),
            jax.ShapeDtypeStruct((_NSLOT,), jnp.int32),
        ],
        scratch_types=[
            pltpu.VMEM((chunk,), jnp.int32),
            pltpu.VMEM((_CAP + L,), jnp.int32),
            pltpu.VMEM((_CAP + L,), jnp.int32),
            pltpu.VMEM((2, D, _RB), jnp.float32),
            pltpu.VMEM((128, D), jnp.float32),
            pltpu.VMEM((128,), jnp.int32),
            pltpu.VMEM((2 * L,), jnp.int32),
            pltpu.VMEM((D, 128), jnp.float32),
            pltpu.VMEM((nw, L), jnp.int32),
            pltpu.SemaphoreType.DMA,
            pltpu.SemaphoreType.DMA,
        ],
        compiler_params=pltpu.CompilerParams(needs_layout_passes=False),
    )
    def collect_kernel(
        idx_hbm,
        table_hbm,
        base_hbm,
        rows_hbm,
        pos_hbm,
        dchunk,
        idxl,
        jl,
        buf,
        ext,
        pbuf,
        jtmp,
        rtmp,
        basev,
        sem0,
        sem1,
    ):
        w = lax.axis_index("s") * nc + lax.axis_index("c")
        lanes = lax.iota(jnp.int32, L)
        neg1 = jnp.full((L,), -1, jnp.int32)

        pltpu.sync_copy(base_hbm, basev)
        bvec = basev[w, pl.ds(0, L)]
        bw = pl.multiple_of(bvec[0], 128)
        bw1 = pl.multiple_of(bvec[1], 128)

        def reset_pbuf():
            for i in range(128 // L):
                pbuf[pl.ds(i * L, L)] = neg1

        reset_pbuf()
        fill_end = jnp.where(w == nw - 1, jnp.int32(_NSLOT), bw1)
        nfill = (fill_end - bw) // 128

        def fill_body(f, carry):
            pltpu.sync_copy(pbuf, pos_hbm.at[pl.ds(bw + f * 128, 128)])
            return carry

        lax.fori_loop(0, nfill, fill_body, 0)

        def emit_append(jscalar, ec, fl):
            plsc.store_scatter(
                pbuf, [jnp.full((L,), ec, jnp.int32)],
                jnp.full((L,), jscalar, jnp.int32), mask=lanes == 0,
            )
            ec = ec + 1

            def flush(fl):
                pltpu.sync_copy(ext, rows_hbm.at[pl.ds(bw + fl * 128, 128)])
                pltpu.sync_copy(pbuf, pos_hbm.at[pl.ds(bw + fl * 128, 128)])
                reset_pbuf()
                return fl + 1

            fl = lax.cond(ec == 128, flush, lambda f: f, fl)
            ec = jnp.where(ec == 128, 0, ec)
            return ec, fl

        def scan_chunk(c, carry):
            pltpu.sync_copy(idx_hbm.at[pl.ds(c * chunk, chunk)], dchunk)

            def scan_vec(g, carry):
                cnt, ec, fl = carry
                v = dchunk[pl.ds(g * L, L)]
                mine = lax.bitwise_and(lax.shift_right_logical(v, 8), nw - 1) == w
                tailm = v >= (nblk - 1) * _RB
                listm = jnp.logical_and(mine, jnp.logical_not(tailm))
                k = plsc.all_reduce_population_count(listm)[0]
                jv = lanes + (c * chunk + g * L)
                in_cap = cnt + k <= _CAP

                @pl.when(jnp.logical_and(k > 0, in_cap))
                def _listed():
                    pv = cnt + plsc.cumsum(listm.astype(jnp.int32)) - 1
                    plsc.store_scatter(idxl, [pv], v, mask=listm)
                    plsc.store_scatter(jl, [pv], jv, mask=listm)

                ovm = jnp.where(in_cap, jnp.logical_and(mine, tailm), mine)
                novm = plsc.all_reduce_population_count(ovm)[0]

                def _overflow(carry):
                    ec, fl = carry
                    mine_i32 = ovm.astype(jnp.int32)
                    for i in range(L):
                        def do_ov(carry, i=i):
                            ec, fl = carry
                            col = pl.multiple_of(
                                lax.bitwise_and(v[i], ~jnp.int32(127)), 128
                            )
                            pltpu.sync_copy(
                                table_hbm.at[pl.ds(0, D), pl.ds(col, 128)], rtmp
                            )
                            l2 = lax.bitwise_and(v[i], 127)
                            l2v = jnp.full((L,), l2, jnp.int32)
                            for kk in range(D // L):
                                ext[ec, pl.ds(kk * L, L)] = plsc.load_gather(
                                    rtmp, [lanes + kk * L, l2v]
                                )
                            return emit_append(jv[i], ec, fl)

                        carry = lax.cond(mine_i32[i] != 0, do_ov, lambda cc: cc, (ec, fl))
                        ec, fl = carry
                    return ec, fl

                ec, fl = lax.cond(novm > 0, _overflow, lambda cc: cc, (ec, fl))
                cnt = jnp.where(in_cap, cnt + k, cnt)
                return cnt, ec, fl

            return lax.fori_loop(0, chunk // L, scan_vec, carry)

        lc, ec, fl = lax.fori_loop(
            0, n_chunks, scan_chunk, (jnp.int32(0), jnp.int32(0), jnp.int32(0))
        )

        plsc.store_scatter(
            idxl, [lc + lanes], jnp.full((L,), _SENTINEL, jnp.int32)
        )
        nlv = (lc + L - 1) // L

        nfull = nblk - 1

        def blk_start(t):
            return pl.multiple_of((w + nw * t) * _RB, 128)

        def issue(t, p_static):
            sem = sem0 if p_static == 0 else sem1

            @pl.when(w + nw * t < nfull)
            def _():
                pltpu.async_copy(
                    table_hbm.at[pl.ds(0, D), pl.ds(blk_start(t), _RB)],
                    buf.at[p_static],
                    sem,
                )

        issue(jnp.int32(0), 0)
        issue(jnp.int32(1), 1)

        def block_body(t, carry):
            def with_parity(p_static, carry):
                sem = sem0 if p_static == 0 else sem1

                @pl.when(w + nw * t < nfull)
                def _wait():
                    pltpu.make_async_copy(
                        table_hbm.at[pl.ds(0, D), pl.ds(0, _RB)],
                        buf.at[p_static],
                        sem,
                    ).wait()

                start = blk_start(t)

                def scan_list(lv, carry):
                    ec, fl = carry
                    lvv = idxl[pl.ds(lv * L, L)]
                    m = lax.shift_right_logical(lvv, 13) == t
                    km = plsc.all_reduce_population_count(m)[0]

                    def matched(carry):
                        ec, fl = carry
                        pv = plsc.cumsum(m.astype(jnp.int32)) - 1
                        plsc.store_scatter(jtmp, [pv], lvv, mask=m)
                        jlv = jl[pl.ds(lv * L, L)]
                        plsc.store_scatter(jtmp, [pv + L], jlv, mask=m)

                        def entry(e, carry):
                            ec, fl = carry
                            espl = jnp.full((L,), e, jnp.int32)
                            ev0 = plsc.load_gather(jtmp, [espl])[0]
                            jev = plsc.load_gather(jtmp, [espl + L])[0]
                            lv2 = jnp.full((L,), ev0 - start, jnp.int32)
                            pspl = jnp.full((L,), p_static, jnp.int32)
                            for kk in range(D // L):
                                ext[ec, pl.ds(kk * L, L)] = plsc.load_gather(
                                    buf, [pspl, lanes + kk * L, lv2]
                                )
                            return emit_append(jev, ec, fl)

                        return lax.fori_loop(0, km, entry, (ec, fl))

                    return lax.cond(km > 0, matched, lambda cc: cc, (ec, fl))

                carry = lax.fori_loop(0, nlv, scan_list, carry)
                issue(t + 2, p_static)
                return carry

            return lax.cond(
                lax.rem(t, 2) == 0,
                lambda cc: with_parity(0, cc),
                lambda cc: with_parity(1, cc),
                carry,
            )

        if _BISECT_PHASE2:
            ec, fl = lax.fori_loop(0, tmax, block_body, (ec, fl))

        @pl.when(ec > 0)
        def _final_flush():
            pltpu.sync_copy(ext, rows_hbm.at[pl.ds(bw + fl * 128, 128)])
            pltpu.sync_copy(pbuf, pos_hbm.at[pl.ds(bw + fl * 128, 128)])

    return collect_kernel


@functools.cache
def _build_unscatter(B: int, D: int):
    info = plsc.get_sparse_core_info()
    nc = info.num_cores
    nw = nc * info.num_subcores
    spw = _NSLOT // nw
    npv = spw // 128

    mesh = plsc.VectorSubcoreMesh(core_axis_name="c", subcore_axis_name="s")

    @functools.partial(
        pl.kernel,
        mesh=mesh,
        out_type=jax.ShapeDtypeStruct((B, D), jnp.float32),
        scratch_types=[
            pltpu.VMEM((npv, 128), jnp.int32),
            pltpu.VMEM((spw, D), jnp.float32),
            pltpu.SemaphoreType.DMA,
        ],
        compiler_params=pltpu.CompilerParams(use_tc_tiling_on_sc=False),
    )
    def unscatter_kernel(rows_hbm, pos_hbm, out_hbm, pv, rv, sem):
        w = lax.axis_index("s") * nc + lax.axis_index("c")
        base = w * spw
        pltpu.sync_copy(pos_hbm.at[pl.ds(w * npv, npv)], pv)
        pltpu.sync_copy(rows_hbm.at[pl.ds(base, spw)], rv)
        copies = []
        for k in range(npv):
            copies.append(
                pltpu.async_copy(
                    rv.at[pl.ds(k * 128, 128)],
                    out_hbm.at[plsc.Indices(pv.at[k], ignored_value=-1)],
                    sem,
                )
            )
        for c in copies:
            c.wait()

    return unscatter_kernel


def kernel(data, emb):
    (B,) = data.shape
    V, D = emb.shape
    nw = 32
    tile_of = lax.bitwise_and(lax.shift_right_logical(data, 8), nw - 1)
    counts = jnp.zeros((nw,), jnp.int32).at[tile_of].add(1)
    regsz = ((counts + 127) // 128) * 128 + _SLACK
    base = jnp.concatenate(
        [jnp.zeros((1,), jnp.int32), jnp.cumsum(regsz, dtype=jnp.int32)]
    )
    base_pairs = jnp.pad(
        jnp.stack([base[:nw], base[1 : nw + 1]], axis=1), ((0, 0), (0, 14))
    )
    rows, pos = _build_collect(B, V, D)(data, emb.T, base_pairs)
    pos2 = pos.reshape(_NSLOT // 128, 128)
    return _build_unscatter(B, D)(rows, pos2)


# speedup vs baseline: 2.4301x; 1.0019x over previous
import functools

import jax
import jax.numpy as jnp
from jax import lax
from jax.experimental import pallas as pl
from jax.experimental.pallas import tpu as pltpu
from jax.experimental.pallas import tpu_sc as plsc

_RB = 256
_CAP = 2048
_SLACK = 0
_NSLOT = 20480
_SENTINEL = 0x7FFFFFF0
_BISECT_PHASE2 = True
_BISECT_OVERFLOW = True


@functools.cache
def _build_collect(B: int, V: int, D: int):
    info = plsc.get_sparse_core_info()
    L = info.num_lanes
    nc = info.num_cores
    nw = nc * info.num_subcores
    nblk = (V + _RB - 1) // _RB
    tmax = (nblk + nw - 1) // nw
    n_chunks = 4
    chunk = B // n_chunks

    mesh = plsc.VectorSubcoreMesh(core_axis_name="c", subcore_axis_name="s")

    @functools.partial(
        pl.kernel,
        mesh=mesh,
        out_type=[
            jax.ShapeDtypeStruct((_NSLOT, D), jnp.float32),
            jax.ShapeDtypeStruct((_NSLOT,), jnp.int32),
        ],
        scratch_types=[
            pltpu.VMEM((chunk,), jnp.int32),
            pltpu.VMEM((_CAP + L,), jnp.int32),
            pltpu.VMEM((_CAP + L,), jnp.int32),
            pltpu.VMEM((2, D, _RB), jnp.float32),
            pltpu.VMEM((128, D), jnp.float32),
            pltpu.VMEM((128,), jnp.int32),
            pltpu.VMEM((2 * L,), jnp.int32),
            pltpu.VMEM((D, 128), jnp.float32),
            pltpu.VMEM((nw, L), jnp.int32),
            pltpu.SemaphoreType.DMA,
            pltpu.SemaphoreType.DMA,
        ],
        compiler_params=pltpu.CompilerParams(needs_layout_passes=False),
    )
    def collect_kernel(
        idx_hbm,
        table_hbm,
        base_hbm,
        rows_hbm,
        pos_hbm,
        dchunk,
        idxl,
        jl,
        buf,
        ext,
        pbuf,
        jtmp,
        rtmp,
        basev,
        sem0,
        sem1,
    ):
        w = lax.axis_index("s") * nc + lax.axis_index("c")
        lanes = lax.iota(jnp.int32, L)
        neg1 = jnp.full((L,), -1, jnp.int32)

        pltpu.sync_copy(base_hbm, basev)
        bvec = basev[w, pl.ds(0, L)]
        bw = pl.multiple_of(bvec[0], 128)
        bw1 = pl.multiple_of(bvec[1], 128)

        def reset_pbuf():
            for i in range(128 // L):
                pbuf[pl.ds(i * L, L)] = neg1

        reset_pbuf()
        fill_end = jnp.where(w == nw - 1, jnp.int32(_NSLOT), bw1)
        nfill = (fill_end - bw) // 128

        def fill_body(f, carry):
            pltpu.sync_copy(pbuf, pos_hbm.at[pl.ds(bw + f * 128, 128)])
            return carry

        lax.fori_loop(0, nfill, fill_body, 0)

        def emit_append(jscalar, ec, fl):
            plsc.store_scatter(
                pbuf, [jnp.full((L,), ec, jnp.int32)],
                jnp.full((L,), jscalar, jnp.int32), mask=lanes == 0,
            )
            ec = ec + 1

            def flush(fl):
                pltpu.sync_copy(ext, rows_hbm.at[pl.ds(bw + fl * 128, 128)])
                pltpu.sync_copy(pbuf, pos_hbm.at[pl.ds(bw + fl * 128, 128)])
                reset_pbuf()
                return fl + 1

            fl = lax.cond(ec == 128, flush, lambda f: f, fl)
            ec = jnp.where(ec == 128, 0, ec)
            return ec, fl

        def scan_chunk(c, carry):
            pltpu.sync_copy(idx_hbm.at[pl.ds(c * chunk, chunk)], dchunk)

            def scan_vec(g, carry):
                cnt, ec, fl = carry
                v = dchunk[pl.ds(g * L, L)]
                mine = lax.bitwise_and(lax.shift_right_logical(v, 8), nw - 1) == w
                tailm = v >= (nblk - 1) * _RB
                listm = jnp.logical_and(mine, jnp.logical_not(tailm))
                k = plsc.all_reduce_population_count(listm)[0]
                jv = lanes + (c * chunk + g * L)
                in_cap = cnt + k <= _CAP

                @pl.when(jnp.logical_and(k > 0, in_cap))
                def _listed():
                    pv = cnt + plsc.cumsum(listm.astype(jnp.int32)) - 1
                    plsc.store_scatter(idxl, [pv], v, mask=listm)
                    plsc.store_scatter(jl, [pv], jv, mask=listm)

                ovm = jnp.where(in_cap, jnp.logical_and(mine, tailm), mine)
                novm = plsc.all_reduce_population_count(ovm)[0]

                def _overflow(carry):
                    ec, fl = carry
                    mine_i32 = ovm.astype(jnp.int32)
                    for i in range(L):
                        def do_ov(carry, i=i):
                            ec, fl = carry
                            col = pl.multiple_of(
                                lax.bitwise_and(v[i], ~jnp.int32(127)), 128
                            )
                            pltpu.sync_copy(
                                table_hbm.at[pl.ds(0, D), pl.ds(col, 128)], rtmp
                            )
                            l2 = lax.bitwise_and(v[i], 127)
                            l2v = jnp.full((L,), l2, jnp.int32)
                            for kk in range(D // L):
                                ext[ec, pl.ds(kk * L, L)] = plsc.load_gather(
                                    rtmp, [lanes + kk * L, l2v]
                                )
                            return emit_append(jv[i], ec, fl)

                        carry = lax.cond(mine_i32[i] != 0, do_ov, lambda cc: cc, (ec, fl))
                        ec, fl = carry
                    return ec, fl

                ec, fl = lax.cond(novm > 0, _overflow, lambda cc: cc, (ec, fl))
                cnt = jnp.where(in_cap, cnt + k, cnt)
                return cnt, ec, fl

            return lax.fori_loop(0, chunk // L, scan_vec, carry)

        lc, ec, fl = lax.fori_loop(
            0, n_chunks, scan_chunk, (jnp.int32(0), jnp.int32(0), jnp.int32(0))
        )

        plsc.store_scatter(
            idxl, [lc + lanes], jnp.full((L,), _SENTINEL, jnp.int32)
        )
        nlv = (lc + L - 1) // L

        nfull = nblk - 1

        def blk_start(t):
            return pl.multiple_of((w + nw * t) * _RB, 128)

        def issue(t, p_static):
            sem = sem0 if p_static == 0 else sem1

            @pl.when(w + nw * t < nfull)
            def _():
                for gg in range(D // 8):
                    pltpu.async_copy(
                        table_hbm.at[pl.ds(gg * 8, 8), pl.ds(blk_start(t), _RB)],
                        buf.at[p_static, pl.ds(gg * 8, 8)],
                        sem,
                    )

        issue(jnp.int32(0), 0)
        issue(jnp.int32(1), 1)

        def block_body(t, carry):
            def with_parity(p_static, carry):
                sem = sem0 if p_static == 0 else sem1

                @pl.when(w + nw * t < nfull)
                def _wait():
                    pltpu.make_async_copy(
                        table_hbm.at[pl.ds(0, D), pl.ds(0, _RB)],
                        buf.at[p_static],
                        sem,
                    ).wait()

                start = blk_start(t)

                def scan_list(lv, carry):
                    ec, fl = carry
                    lvv = idxl[pl.ds(lv * L, L)]
                    m = lax.shift_right_logical(lvv, 13) == t
                    km = plsc.all_reduce_population_count(m)[0]

                    def matched(carry):
                        ec, fl = carry
                        pv = plsc.cumsum(m.astype(jnp.int32)) - 1
                        plsc.store_scatter(jtmp, [pv], lvv, mask=m)
                        jlv = jl[pl.ds(lv * L, L)]
                        plsc.store_scatter(jtmp, [pv + L], jlv, mask=m)

                        def entry(e, carry):
                            ec, fl = carry
                            espl = jnp.full((L,), e, jnp.int32)
                            ev0 = plsc.load_gather(jtmp, [espl])[0]
                            jev = plsc.load_gather(jtmp, [espl + L])[0]
                            lv2 = jnp.full((L,), ev0 - start, jnp.int32)
                            pspl = jnp.full((L,), p_static, jnp.int32)
                            for kk in range(D // L):
                                ext[ec, pl.ds(kk * L, L)] = plsc.load_gather(
                                    buf, [pspl, lanes + kk * L, lv2]
                                )
                            return emit_append(jev, ec, fl)

                        return lax.fori_loop(0, km, entry, (ec, fl))

                    return lax.cond(km > 0, matched, lambda cc: cc, (ec, fl))

                carry = lax.fori_loop(0, nlv, scan_list, carry)
                issue(t + 2, p_static)
                return carry

            return lax.cond(
                lax.rem(t, 2) == 0,
                lambda cc: with_parity(0, cc),
                lambda cc: with_parity(1, cc),
                carry,
            )

        if _BISECT_PHASE2:
            ec, fl = lax.fori_loop(0, tmax, block_body, (ec, fl))

        @pl.when(ec > 0)
        def _final_flush():
            pltpu.sync_copy(ext, rows_hbm.at[pl.ds(bw + fl * 128, 128)])
            pltpu.sync_copy(pbuf, pos_hbm.at[pl.ds(bw + fl * 128, 128)])

    return collect_kernel


@functools.cache
def _build_unscatter(B: int, D: int):
    info = plsc.get_sparse_core_info()
    nc = info.num_cores
    nw = nc * info.num_subcores
    spw = _NSLOT // nw
    npv = spw // 128

    mesh = plsc.VectorSubcoreMesh(core_axis_name="c", subcore_axis_name="s")

    @functools.partial(
        pl.kernel,
        mesh=mesh,
        out_type=jax.ShapeDtypeStruct((B, D), jnp.float32),
        scratch_types=[
            pltpu.VMEM((npv, 128), jnp.int32),
            pltpu.VMEM((spw, D), jnp.float32),
            pltpu.SemaphoreType.DMA,
        ],
        compiler_params=pltpu.CompilerParams(use_tc_tiling_on_sc=False),
    )
    def unscatter_kernel(rows_hbm, pos_hbm, out_hbm, pv, rv, sem):
        w = lax.axis_index("s") * nc + lax.axis_index("c")
        base = w * spw
        pltpu.sync_copy(pos_hbm.at[pl.ds(w * npv, npv)], pv)
        pltpu.sync_copy(rows_hbm.at[pl.ds(base, spw)], rv)
        copies = []
        for k in range(npv):
            copies.append(
                pltpu.async_copy(
                    rv.at[pl.ds(k * 128, 128)],
                    out_hbm.at[plsc.Indices(pv.at[k], ignored_value=-1)],
                    sem,
                )
            )
        for c in copies:
            c.wait()

    return unscatter_kernel


def kernel(data, emb):
    (B,) = data.shape
    V, D = emb.shape
    nw = 32
    tile_of = lax.bitwise_and(lax.shift_right_logical(data, 8), nw - 1)
    counts = jnp.zeros((nw,), jnp.int32).at[tile_of].add(1)
    regsz = ((counts + 127) // 128) * 128 + _SLACK
    base = jnp.concatenate(
        [jnp.zeros((1,), jnp.int32), jnp.cumsum(regsz, dtype=jnp.int32)]
    )
    base_pairs = jnp.pad(
        jnp.stack([base[:nw], base[1 : nw + 1]], axis=1), ((0, 0), (0, 14))
    )
    rows, pos = _build_collect(B, V, D)(data, emb.T, base_pairs)
    pos2 = pos.reshape(_NSLOT // 128, 128)
    return _build_unscatter(B, D)(rows, pos2)


# speedup vs baseline: 2.4473x; 1.0071x over previous
import functools

import jax
import jax.numpy as jnp
from jax import lax
from jax.experimental import pallas as pl
from jax.experimental.pallas import tpu as pltpu
from jax.experimental.pallas import tpu_sc as plsc

_RB = 256
_CAP = 2048
_SLACK = 0
_NSLOT = 20480
_SENTINEL = 0x7FFFFFF0
_BISECT_PHASE2 = True
_BISECT_OVERFLOW = True
_DIAG_NOSCAN = False


@functools.cache
def _build_collect(B: int, V: int, D: int):
    info = plsc.get_sparse_core_info()
    L = info.num_lanes
    nc = info.num_cores
    nw = nc * info.num_subcores
    nblk = (V + _RB - 1) // _RB
    tmax = (nblk + nw - 1) // nw
    n_chunks = 4
    chunk = B // n_chunks

    mesh = plsc.VectorSubcoreMesh(core_axis_name="c", subcore_axis_name="s")

    @functools.partial(
        pl.kernel,
        mesh=mesh,
        out_type=[
            jax.ShapeDtypeStruct((_NSLOT, D), jnp.float32),
            jax.ShapeDtypeStruct((_NSLOT,), jnp.int32),
        ],
        scratch_types=[
            pltpu.VMEM((chunk,), jnp.int32),
            pltpu.VMEM((_CAP + L,), jnp.int32),
            pltpu.VMEM((_CAP + L,), jnp.int32),
            pltpu.VMEM((_CAP + L,), jnp.int32),
            pltpu.VMEM((2, D, _RB), jnp.float32),
            pltpu.VMEM((128, D), jnp.float32),
            pltpu.VMEM((128,), jnp.int32),
            pltpu.VMEM((2 * L,), jnp.int32),
            pltpu.VMEM((D, 128), jnp.float32),
            pltpu.VMEM((nw, L), jnp.int32),
            pltpu.SemaphoreType.DMA,
            pltpu.SemaphoreType.DMA,
        ],
        compiler_params=pltpu.CompilerParams(needs_layout_passes=False),
    )
    def collect_kernel(
        idx_hbm,
        table_hbm,
        base_hbm,
        rows_hbm,
        pos_hbm,
        dchunk,
        idxl,
        jl,
        tl,
        buf,
        ext,
        pbuf,
        jtmp,
        rtmp,
        basev,
        sem0,
        sem1,
    ):
        w = lax.axis_index("s") * nc + lax.axis_index("c")
        lanes = lax.iota(jnp.int32, L)
        neg1 = jnp.full((L,), -1, jnp.int32)

        pltpu.sync_copy(base_hbm, basev)
        bvec = basev[w, pl.ds(0, L)]
        bw = pl.multiple_of(bvec[0], 128)
        bw1 = pl.multiple_of(bvec[1], 128)

        def reset_pbuf():
            for i in range(128 // L):
                pbuf[pl.ds(i * L, L)] = neg1

        reset_pbuf()
        fill_end = jnp.where(w == nw - 1, jnp.int32(_NSLOT), bw1)
        nfill = (fill_end - bw) // 128

        def fill_body(f, carry):
            pltpu.sync_copy(pbuf, pos_hbm.at[pl.ds(bw + f * 128, 128)])
            return carry

        lax.fori_loop(0, nfill, fill_body, 0)

        def emit_append(jscalar, ec, fl):
            plsc.store_scatter(
                pbuf, [jnp.full((L,), ec, jnp.int32)],
                jnp.full((L,), jscalar, jnp.int32), mask=lanes == 0,
            )
            ec = ec + 1

            def flush(fl):
                pltpu.sync_copy(ext, rows_hbm.at[pl.ds(bw + fl * 128, 128)])
                pltpu.sync_copy(pbuf, pos_hbm.at[pl.ds(bw + fl * 128, 128)])
                reset_pbuf()
                return fl + 1

            fl = lax.cond(ec == 128, flush, lambda f: f, fl)
            ec = jnp.where(ec == 128, 0, ec)
            return ec, fl

        def scan_chunk(c, carry):
            pltpu.sync_copy(idx_hbm.at[pl.ds(c * chunk, chunk)], dchunk)

            def scan_vec(g, carry):
                cnt, ec, fl = carry
                v = dchunk[pl.ds(g * L, L)]
                mine = lax.bitwise_and(lax.shift_right_logical(v, 8), nw - 1) == w
                tailm = v >= (nblk - 1) * _RB
                listm = jnp.logical_and(mine, jnp.logical_not(tailm))
                k = plsc.all_reduce_population_count(listm)[0]
                jv = lanes + (c * chunk + g * L)
                in_cap = cnt + k <= _CAP

                @pl.when(jnp.logical_and(k > 0, in_cap))
                def _listed():
                    pv = cnt + plsc.cumsum(listm.astype(jnp.int32)) - 1
                    plsc.store_scatter(idxl, [pv], v, mask=listm)
                    plsc.store_scatter(jl, [pv], jv, mask=listm)

                ovm = jnp.where(in_cap, jnp.logical_and(mine, tailm), mine)
                novm = plsc.all_reduce_population_count(ovm)[0]

                def _overflow(carry):
                    ec, fl = carry
                    mine_i32 = ovm.astype(jnp.int32)
                    for i in range(L):
                        def do_ov(carry, i=i):
                            ec, fl = carry
                            col = pl.multiple_of(
                                lax.bitwise_and(v[i], ~jnp.int32(127)), 128
                            )
                            pltpu.sync_copy(
                                table_hbm.at[pl.ds(0, D), pl.ds(col, 128)], rtmp
                            )
                            l2 = lax.bitwise_and(v[i], 127)
                            l2v = jnp.full((L,), l2, jnp.int32)
                            for kk in range(D // L):
                                ext[ec, pl.ds(kk * L, L)] = plsc.load_gather(
                                    rtmp, [lanes + kk * L, l2v]
                                )
                            return emit_append(jv[i], ec, fl)

                        carry = lax.cond(mine_i32[i] != 0, do_ov, lambda cc: cc, (ec, fl))
                        ec, fl = carry
                    return ec, fl

                ec, fl = lax.cond(novm > 0, _overflow, lambda cc: cc, (ec, fl))
                cnt = jnp.where(in_cap, cnt + k, cnt)
                return cnt, ec, fl

            return lax.fori_loop(0, chunk // L, scan_vec, carry)

        lc, ec, fl = lax.fori_loop(
            0, n_chunks, scan_chunk, (jnp.int32(0), jnp.int32(0), jnp.int32(0))
        )

        plsc.store_scatter(
            idxl, [lc + lanes], jnp.full((L,), _SENTINEL, jnp.int32)
        )
        nlv = (lc + L - 1) // L

        def tconv(i, carry):
            tl[pl.ds(i * L, L)] = lax.shift_right_logical(idxl[pl.ds(i * L, L)], 13)
            return carry

        lax.fori_loop(0, nlv + 1, tconv, 0)

        nfull = nblk - 1

        def blk_start(t):
            return pl.multiple_of((w + nw * t) * _RB, 128)

        def issue(t, p_static):
            sem = sem0 if p_static == 0 else sem1

            @pl.when(w + nw * t < nfull)
            def _():
                for gg in range(D // 8):
                    pltpu.async_copy(
                        table_hbm.at[pl.ds(gg * 8, 8), pl.ds(blk_start(t), _RB)],
                        buf.at[p_static, pl.ds(gg * 8, 8)],
                        sem,
                    )

        issue(jnp.int32(0), 0)
        issue(jnp.int32(1), 1)

        def block_body(t, carry):
            def with_parity(p_static, carry):
                sem = sem0 if p_static == 0 else sem1

                @pl.when(w + nw * t < nfull)
                def _wait():
                    pltpu.make_async_copy(
                        table_hbm.at[pl.ds(0, D), pl.ds(0, _RB)],
                        buf.at[p_static],
                        sem,
                    ).wait()

                start = blk_start(t)

                def scan_list(lv, carry):
                    ec, fl = carry
                    m = tl[pl.ds(lv * L, L)] == t
                    km = plsc.all_reduce_population_count(m)[0]

                    def matched(carry):
                        ec, fl = carry
                        lvv = idxl[pl.ds(lv * L, L)]
                        pv = plsc.cumsum(m.astype(jnp.int32)) - 1
                        plsc.store_scatter(jtmp, [pv], lvv, mask=m)
                        jlv = jl[pl.ds(lv * L, L)]
                        plsc.store_scatter(jtmp, [pv + L], jlv, mask=m)

                        def entry(e, carry):
                            ec, fl = carry
                            espl = jnp.full((L,), e, jnp.int32)
                            ev0 = plsc.load_gather(jtmp, [espl])[0]
                            jev = plsc.load_gather(jtmp, [espl + L])[0]
                            lv2 = jnp.full((L,), ev0 - start, jnp.int32)
                            pspl = jnp.full((L,), p_static, jnp.int32)
                            for kk in range(D // L):
                                ext[ec, pl.ds(kk * L, L)] = plsc.load_gather(
                                    buf, [pspl, lanes + kk * L, lv2]
                                )
                            return emit_append(jev, ec, fl)

                        return lax.fori_loop(0, km, entry, (ec, fl))

                    return lax.cond(km > 0, matched, lambda cc: cc, (ec, fl))

                if not _DIAG_NOSCAN:
                    carry = lax.fori_loop(0, nlv, scan_list, carry)
                issue(t + 2, p_static)
                return carry

            return lax.cond(
                lax.rem(t, 2) == 0,
                lambda cc: with_parity(0, cc),
                lambda cc: with_parity(1, cc),
                carry,
            )

        if _BISECT_PHASE2:
            ec, fl = lax.fori_loop(0, tmax, block_body, (ec, fl))

        @pl.when(ec > 0)
        def _final_flush():
            pltpu.sync_copy(ext, rows_hbm.at[pl.ds(bw + fl * 128, 128)])
            pltpu.sync_copy(pbuf, pos_hbm.at[pl.ds(bw + fl * 128, 128)])

    return collect_kernel


@functools.cache
def _build_unscatter(B: int, D: int):
    info = plsc.get_sparse_core_info()
    nc = info.num_cores
    nw = nc * info.num_subcores
    spw = _NSLOT // nw
    npv = spw // 128

    mesh = plsc.VectorSubcoreMesh(core_axis_name="c", subcore_axis_name="s")

    @functools.partial(
        pl.kernel,
        mesh=mesh,
        out_type=jax.ShapeDtypeStruct((B, D), jnp.float32),
        scratch_types=[
            pltpu.VMEM((npv, 128), jnp.int32),
            pltpu.VMEM((spw, D), jnp.float32),
            pltpu.SemaphoreType.DMA,
        ],
        compiler_params=pltpu.CompilerParams(use_tc_tiling_on_sc=False),
    )
    def unscatter_kernel(rows_hbm, pos_hbm, out_hbm, pv, rv, sem):
        w = lax.axis_index("s") * nc + lax.axis_index("c")
        base = w * spw
        pltpu.sync_copy(pos_hbm.at[pl.ds(w * npv, npv)], pv)
        pltpu.sync_copy(rows_hbm.at[pl.ds(base, spw)], rv)
        copies = []
        for k in range(npv):
            copies.append(
                pltpu.async_copy(
                    rv.at[pl.ds(k * 128, 128)],
                    out_hbm.at[plsc.Indices(pv.at[k], ignored_value=-1)],
                    sem,
                )
            )
        for c in copies:
            c.wait()

    return unscatter_kernel


def kernel(data, emb):
    (B,) = data.shape
    V, D = emb.shape
    nw = 32
    tile_of = lax.bitwise_and(lax.shift_right_logical(data, 8), nw - 1)
    counts = jnp.zeros((nw,), jnp.int32).at[tile_of].add(1)
    regsz = ((counts + 127) // 128) * 128 + _SLACK
    base = jnp.concatenate(
        [jnp.zeros((1,), jnp.int32), jnp.cumsum(regsz, dtype=jnp.int32)]
    )
    base_pairs = jnp.pad(
        jnp.stack([base[:nw], base[1 : nw + 1]], axis=1), ((0, 0), (0, 14))
    )
    rows, pos = _build_collect(B, V, D)(data, emb.T, base_pairs)
    pos2 = pos.reshape(_NSLOT // 128, 128)
    return _build_unscatter(B, D)(rows, pos2)


# speedup vs baseline: 2.5082x; 1.0249x over previous
import functools

import jax
import jax.numpy as jnp
from jax import lax
from jax.experimental import pallas as pl
from jax.experimental.pallas import tpu as pltpu
from jax.experimental.pallas import tpu_sc as plsc

_RB = 256
_CAP = 1024
_SLACK = 0
_NSLOT = 20480
_SENTINEL = 0x7FFFFFF0
_BISECT_PHASE2 = True
_BISECT_OVERFLOW = True
_DIAG_NOSCAN = False


@functools.cache
def _build_collect(B: int, V: int, D: int):
    info = plsc.get_sparse_core_info()
    L = info.num_lanes
    nc = info.num_cores
    nw = nc * info.num_subcores
    nblk = (V + _RB - 1) // _RB
    tmax = (nblk + nw - 1) // nw
    n_chunks = 8
    chunk = B // n_chunks

    mesh = plsc.VectorSubcoreMesh(core_axis_name="c", subcore_axis_name="s")

    @functools.partial(
        pl.kernel,
        mesh=mesh,
        out_type=[
            jax.ShapeDtypeStruct((_NSLOT, D), jnp.float32),
            jax.ShapeDtypeStruct((_NSLOT,), jnp.int32),
        ],
        scratch_types=[
            pltpu.VMEM((chunk,), jnp.int32),
            pltpu.VMEM((_CAP + L,), jnp.int32),
            pltpu.VMEM((_CAP + L,), jnp.int32),
            pltpu.VMEM((_CAP + L,), jnp.int32),
            pltpu.VMEM((3, D, _RB), jnp.float32),
            pltpu.VMEM((64, D), jnp.float32),
            pltpu.VMEM((64,), jnp.int32),
            pltpu.VMEM((2 * L,), jnp.int32),
            pltpu.VMEM((D, 128), jnp.float32),
            pltpu.VMEM((nw, L), jnp.int32),
            pltpu.SemaphoreType.DMA,
            pltpu.SemaphoreType.DMA,
            pltpu.SemaphoreType.DMA,
        ],
        compiler_params=pltpu.CompilerParams(needs_layout_passes=False),
    )
    def collect_kernel(
        idx_hbm,
        table_hbm,
        base_hbm,
        rows_hbm,
        pos_hbm,
        dchunk,
        idxl,
        jl,
        tl,
        buf,
        ext,
        pbuf,
        jtmp,
        rtmp,
        basev,
        sem0,
        sem1,
        sem2,
    ):
        w = lax.axis_index("s") * nc + lax.axis_index("c")
        lanes = lax.iota(jnp.int32, L)
        neg1 = jnp.full((L,), -1, jnp.int32)

        pltpu.sync_copy(base_hbm, basev)
        bvec = basev[w, pl.ds(0, L)]
        bw = pl.multiple_of(bvec[0], 128)
        bw1 = pl.multiple_of(bvec[1], 128)

        def reset_pbuf():
            for i in range(64 // L):
                pbuf[pl.ds(i * L, L)] = neg1

        reset_pbuf()
        fill_end = jnp.where(w == nw - 1, jnp.int32(_NSLOT), bw1)
        nfill = (fill_end - bw) // 64

        def fill_body(f, carry):
            pltpu.sync_copy(pbuf, pos_hbm.at[pl.ds(bw + f * 64, 64)])
            return carry

        lax.fori_loop(0, nfill, fill_body, 0)

        def emit_append(jscalar, ec, fl):
            plsc.store_scatter(
                pbuf, [jnp.full((L,), ec, jnp.int32)],
                jnp.full((L,), jscalar, jnp.int32), mask=lanes == 0,
            )
            ec = ec + 1

            def flush(fl):
                pltpu.sync_copy(ext, rows_hbm.at[pl.ds(bw + fl * 64, 64)])
                pltpu.sync_copy(pbuf, pos_hbm.at[pl.ds(bw + fl * 64, 64)])
                reset_pbuf()
                return fl + 1

            fl = lax.cond(ec == 64, flush, lambda f: f, fl)
            ec = jnp.where(ec == 64, 0, ec)
            return ec, fl

        def scan_chunk(c, carry):
            pltpu.sync_copy(idx_hbm.at[pl.ds(c * chunk, chunk)], dchunk)

            def scan_vec(g, carry):
                cnt, ec, fl = carry
                v = dchunk[pl.ds(g * L, L)]
                mine = lax.bitwise_and(lax.shift_right_logical(v, 8), nw - 1) == w
                tailm = v >= (nblk - 1) * _RB
                listm = jnp.logical_and(mine, jnp.logical_not(tailm))
                k = plsc.all_reduce_population_count(listm)[0]
                jv = lanes + (c * chunk + g * L)
                in_cap = cnt + k <= _CAP

                @pl.when(jnp.logical_and(k > 0, in_cap))
                def _listed():
                    pv = cnt + plsc.cumsum(listm.astype(jnp.int32)) - 1
                    plsc.store_scatter(idxl, [pv], v, mask=listm)
                    plsc.store_scatter(jl, [pv], jv, mask=listm)

                ovm = jnp.where(in_cap, jnp.logical_and(mine, tailm), mine)
                novm = plsc.all_reduce_population_count(ovm)[0]

                def _overflow(carry):
                    ec, fl = carry
                    mine_i32 = ovm.astype(jnp.int32)
                    for i in range(L):
                        def do_ov(carry, i=i):
                            ec, fl = carry
                            col = pl.multiple_of(
                                lax.bitwise_and(v[i], ~jnp.int32(127)), 128
                            )
                            pltpu.sync_copy(
                                table_hbm.at[pl.ds(0, D), pl.ds(col, 128)], rtmp
                            )
                            l2 = lax.bitwise_and(v[i], 127)
                            l2v = jnp.full((L,), l2, jnp.int32)
                            for kk in range(D // L):
                                ext[ec, pl.ds(kk * L, L)] = plsc.load_gather(
                                    rtmp, [lanes + kk * L, l2v]
                                )
                            return emit_append(jv[i], ec, fl)

                        carry = lax.cond(mine_i32[i] != 0, do_ov, lambda cc: cc, (ec, fl))
                        ec, fl = carry
                    return ec, fl

                ec, fl = lax.cond(novm > 0, _overflow, lambda cc: cc, (ec, fl))
                cnt = jnp.where(in_cap, cnt + k, cnt)
                return cnt, ec, fl

            return lax.fori_loop(0, chunk // L, scan_vec, carry)

        lc, ec, fl = lax.fori_loop(
            0, n_chunks, scan_chunk, (jnp.int32(0), jnp.int32(0), jnp.int32(0))
        )

        plsc.store_scatter(
            idxl, [lc + lanes], jnp.full((L,), _SENTINEL, jnp.int32)
        )
        nlv = (lc + L - 1) // L

        def tconv(i, carry):
            tl[pl.ds(i * L, L)] = lax.shift_right_logical(idxl[pl.ds(i * L, L)], 13)
            return carry

        lax.fori_loop(0, nlv + 1, tconv, 0)

        nfull = nblk - 1

        def blk_start(t):
            return pl.multiple_of((w + nw * t) * _RB, 128)

        sems = (sem0, sem1, sem2)

        def issue(t, p_static):
            sem = sems[p_static]

            @pl.when(w + nw * t < nfull)
            def _():
                for gg in range(D // 8):
                    pltpu.async_copy(
                        table_hbm.at[pl.ds(gg * 8, 8), pl.ds(blk_start(t), _RB)],
                        buf.at[p_static, pl.ds(gg * 8, 8)],
                        sem,
                    )

        issue(jnp.int32(0), 0)
        issue(jnp.int32(1), 1)
        issue(jnp.int32(2), 2)

        def block_body(t, carry):
            def with_parity(p_static, carry):
                sem = sems[p_static]

                @pl.when(w + nw * t < nfull)
                def _wait():
                    pltpu.make_async_copy(
                        table_hbm.at[pl.ds(0, D), pl.ds(0, _RB)],
                        buf.at[p_static],
                        sem,
                    ).wait()

                issue(t + 3, (p_static + 3) % 3)

                start = blk_start(t)

                def scan_list(lv, carry):
                    ec, fl = carry
                    m = tl[pl.ds(lv * L, L)] == t
                    km = plsc.all_reduce_population_count(m)[0]

                    def matched(carry):
                        ec, fl = carry
                        lvv = idxl[pl.ds(lv * L, L)]
                        pv = plsc.cumsum(m.astype(jnp.int32)) - 1
                        plsc.store_scatter(jtmp, [pv], lvv, mask=m)
                        jlv = jl[pl.ds(lv * L, L)]
                        plsc.store_scatter(jtmp, [pv + L], jlv, mask=m)

                        def entry(e, carry):
                            ec, fl = carry
                            espl = jnp.full((L,), e, jnp.int32)
                            ev0 = plsc.load_gather(jtmp, [espl])[0]
                            jev = plsc.load_gather(jtmp, [espl + L])[0]
                            lv2 = jnp.full((L,), ev0 - start, jnp.int32)
                            pspl = jnp.full((L,), p_static, jnp.int32)
                            for kk in range(D // L):
                                ext[ec, pl.ds(kk * L, L)] = plsc.load_gather(
                                    buf, [pspl, lanes + kk * L, lv2]
                                )
                            return emit_append(jev, ec, fl)

                        return lax.fori_loop(0, km, entry, (ec, fl))

                    return lax.cond(km > 0, matched, lambda cc: cc, (ec, fl))

                if not _DIAG_NOSCAN:
                    carry = lax.fori_loop(0, nlv, scan_list, carry)
                return carry

            r3 = lax.rem(t, 3)
            return lax.cond(
                r3 == 0,
                lambda cc: with_parity(0, cc),
                lambda cc: lax.cond(
                    r3 == 1,
                    lambda c2: with_parity(1, c2),
                    lambda c2: with_parity(2, c2),
                    cc,
                ),
                carry,
            )

        if _BISECT_PHASE2:
            ec, fl = lax.fori_loop(0, tmax, block_body, (ec, fl))

        @pl.when(ec > 0)
        def _final_flush():
            pltpu.sync_copy(ext, rows_hbm.at[pl.ds(bw + fl * 64, 64)])
            pltpu.sync_copy(pbuf, pos_hbm.at[pl.ds(bw + fl * 64, 64)])

    return collect_kernel


@functools.cache
def _build_unscatter(B: int, D: int):
    info = plsc.get_sparse_core_info()
    nc = info.num_cores
    nw = nc * info.num_subcores
    spw = _NSLOT // nw
    npv = spw // 128

    mesh = plsc.VectorSubcoreMesh(core_axis_name="c", subcore_axis_name="s")

    @functools.partial(
        pl.kernel,
        mesh=mesh,
        out_type=jax.ShapeDtypeStruct((B, D), jnp.float32),
        scratch_types=[
            pltpu.VMEM((npv, 128), jnp.int32),
            pltpu.VMEM((spw, D), jnp.float32),
            pltpu.SemaphoreType.DMA,
        ],
        compiler_params=pltpu.CompilerParams(use_tc_tiling_on_sc=False),
    )
    def unscatter_kernel(rows_hbm, pos_hbm, out_hbm, pv, rv, sem):
        w = lax.axis_index("s") * nc + lax.axis_index("c")
        base = w * spw
        pltpu.sync_copy(pos_hbm.at[pl.ds(w * npv, npv)], pv)
        pltpu.sync_copy(rows_hbm.at[pl.ds(base, spw)], rv)
        copies = []
        for k in range(npv):
            copies.append(
                pltpu.async_copy(
                    rv.at[pl.ds(k * 128, 128)],
                    out_hbm.at[plsc.Indices(pv.at[k], ignored_value=-1)],
                    sem,
                )
            )
        for c in copies:
            c.wait()

    return unscatter_kernel


def kernel(data, emb):
    (B,) = data.shape
    V, D = emb.shape
    nw = 32
    tile_of = lax.bitwise_and(lax.shift_right_logical(data, 8), nw - 1)
    counts = jnp.zeros((nw,), jnp.int32).at[tile_of].add(1)
    regsz = ((counts + 127) // 128) * 128 + _SLACK
    base = jnp.concatenate(
        [jnp.zeros((1,), jnp.int32), jnp.cumsum(regsz, dtype=jnp.int32)]
    )
    base_pairs = jnp.pad(
        jnp.stack([base[:nw], base[1 : nw + 1]], axis=1), ((0, 0), (0, 14))
    )
    rows, pos = _build_collect(B, V, D)(data, emb.T, base_pairs)
    pos2 = pos.reshape(_NSLOT // 128, 128)
    return _build_unscatter(B, D)(rows, pos2)


# speedup vs baseline: 2.5132x; 1.0020x over previous
import functools

import jax
import jax.numpy as jnp
from jax import lax
from jax.experimental import pallas as pl
from jax.experimental.pallas import tpu as pltpu
from jax.experimental.pallas import tpu_sc as plsc

_RB = 256
_CAP = 1024
_SLACK = 0
_NSLOT = 20480
_SENTINEL = 0x7FFFFFF0
_BISECT_PHASE2 = True
_BISECT_OVERFLOW = True
_DIAG_NOSCAN = False


@functools.cache
def _build_collect(B: int, V: int, D: int):
    info = plsc.get_sparse_core_info()
    L = info.num_lanes
    nc = info.num_cores
    nw = nc * info.num_subcores
    nblk = (V + _RB - 1) // _RB
    tmax = (nblk + nw - 1) // nw
    n_chunks = 8
    chunk = B // n_chunks

    mesh = plsc.VectorSubcoreMesh(core_axis_name="c", subcore_axis_name="s")

    @functools.partial(
        pl.kernel,
        mesh=mesh,
        out_type=[
            jax.ShapeDtypeStruct((_NSLOT, D), jnp.float32),
            jax.ShapeDtypeStruct((_NSLOT,), jnp.int32),
        ],
        scratch_types=[
            pltpu.VMEM((chunk,), jnp.int32),
            pltpu.VMEM((_CAP + L,), jnp.int32),
            pltpu.VMEM((_CAP + L,), jnp.int32),
            pltpu.VMEM((_CAP + L,), jnp.int32),
            pltpu.VMEM((3, D, _RB), jnp.float32),
            pltpu.VMEM((64, D), jnp.float32),
            pltpu.VMEM((64,), jnp.int32),
            pltpu.VMEM((2 * L,), jnp.int32),
            pltpu.VMEM((D, 128), jnp.float32),
            pltpu.VMEM((nw, L), jnp.int32),
            pltpu.SemaphoreType.DMA,
            pltpu.SemaphoreType.DMA,
            pltpu.SemaphoreType.DMA,
        ],
        compiler_params=pltpu.CompilerParams(needs_layout_passes=False),
    )
    def collect_kernel(
        idx_hbm,
        table_hbm,
        base_hbm,
        rows_hbm,
        pos_hbm,
        dchunk,
        idxl,
        jl,
        tl,
        buf,
        ext,
        pbuf,
        jtmp,
        rtmp,
        basev,
        sem0,
        sem1,
        sem2,
    ):
        w = lax.axis_index("s") * nc + lax.axis_index("c")
        lanes = lax.iota(jnp.int32, L)
        neg1 = jnp.full((L,), -1, jnp.int32)

        pltpu.sync_copy(base_hbm, basev)
        bvec = basev[w, pl.ds(0, L)]
        bw = pl.multiple_of(bvec[0], 128)
        bw1 = pl.multiple_of(bvec[1], 128)

        def reset_pbuf():
            for i in range(64 // L):
                pbuf[pl.ds(i * L, L)] = neg1

        reset_pbuf()
        fill_end = jnp.where(w == nw - 1, jnp.int32(_NSLOT), bw1)
        nfill = (fill_end - bw) // 64

        def fill_body(f, carry):
            pltpu.sync_copy(pbuf, pos_hbm.at[pl.ds(bw + f * 64, 64)])
            return carry

        lax.fori_loop(0, nfill, fill_body, 0)

        def emit_append(jscalar, ec, fl):
            plsc.store_scatter(
                pbuf, [jnp.full((L,), ec, jnp.int32)],
                jnp.full((L,), jscalar, jnp.int32), mask=lanes == 0,
            )
            ec = ec + 1

            def flush(fl):
                pltpu.sync_copy(ext, rows_hbm.at[pl.ds(bw + fl * 64, 64)])
                pltpu.sync_copy(pbuf, pos_hbm.at[pl.ds(bw + fl * 64, 64)])
                reset_pbuf()
                return fl + 1

            fl = lax.cond(ec == 64, flush, lambda f: f, fl)
            ec = jnp.where(ec == 64, 0, ec)
            return ec, fl

        def scan_chunk(c, carry):
            pltpu.sync_copy(idx_hbm.at[pl.ds(c * chunk, chunk)], dchunk)

            def scan_vec(g, carry):
                cnt, ec, fl = carry
                v = dchunk[pl.ds(g * L, L)]
                mine = lax.bitwise_and(lax.shift_right_logical(v, 8), nw - 1) == w
                tailm = v >= (nblk - 1) * _RB
                listm = jnp.logical_and(mine, jnp.logical_not(tailm))
                k = plsc.all_reduce_population_count(listm)[0]
                jv = lanes + (c * chunk + g * L)
                in_cap = cnt + k <= _CAP

                @pl.when(jnp.logical_and(k > 0, in_cap))
                def _listed():
                    pv = cnt + plsc.cumsum(listm.astype(jnp.int32)) - 1
                    plsc.store_scatter(idxl, [pv], v, mask=listm)
                    plsc.store_scatter(jl, [pv], jv, mask=listm)

                ovm = jnp.where(in_cap, jnp.logical_and(mine, tailm), mine)
                novm = plsc.all_reduce_population_count(ovm)[0]

                def _overflow(carry):
                    ec, fl = carry
                    mine_i32 = ovm.astype(jnp.int32)
                    for i in range(L):
                        def do_ov(carry, i=i):
                            ec, fl = carry
                            col = pl.multiple_of(
                                lax.bitwise_and(v[i], ~jnp.int32(127)), 128
                            )
                            pltpu.sync_copy(
                                table_hbm.at[pl.ds(0, D), pl.ds(col, 128)], rtmp
                            )
                            l2 = lax.bitwise_and(v[i], 127)
                            l2v = jnp.full((L,), l2, jnp.int32)
                            for kk in range(D // L):
                                ext[ec, pl.ds(kk * L, L)] = plsc.load_gather(
                                    rtmp, [lanes + kk * L, l2v]
                                )
                            return emit_append(jv[i], ec, fl)

                        carry = lax.cond(mine_i32[i] != 0, do_ov, lambda cc: cc, (ec, fl))
                        ec, fl = carry
                    return ec, fl

                ec, fl = lax.cond(novm > 0, _overflow, lambda cc: cc, (ec, fl))
                cnt = jnp.where(in_cap, cnt + k, cnt)
                return cnt, ec, fl

            return lax.fori_loop(0, chunk // L, scan_vec, carry)

        lc, ec, fl = lax.fori_loop(
            0, n_chunks, scan_chunk, (jnp.int32(0), jnp.int32(0), jnp.int32(0))
        )

        plsc.store_scatter(
            idxl, [lc + lanes], jnp.full((L,), _SENTINEL, jnp.int32)
        )
        nlv = (lc + L - 1) // L

        def tconv(i, carry):
            tl[pl.ds(i * L, L)] = lax.shift_right_logical(idxl[pl.ds(i * L, L)], 13)
            return carry

        lax.fori_loop(0, nlv + 1, tconv, 0)

        nfull = nblk - 1

        def blk_start(t):
            return pl.multiple_of((w + nw * t) * _RB, 128)

        sems = (sem0, sem1, sem2)

        def issue(t, p_static):
            sem = sems[p_static]

            @pl.when(w + nw * t < nfull)
            def _():
                for gg in range(D // 8):
                    pltpu.async_copy(
                        table_hbm.at[pl.ds(gg * 8, 8), pl.ds(blk_start(t), _RB)],
                        buf.at[p_static, pl.ds(gg * 8, 8)],
                        sem,
                    )

        issue(jnp.int32(0), 0)
        issue(jnp.int32(1), 1)

        def block_body(t, carry):
            def with_parity(p_static, carry):
                sem = sems[p_static]

                @pl.when(w + nw * t < nfull)
                def _wait():
                    pltpu.make_async_copy(
                        table_hbm.at[pl.ds(0, D), pl.ds(0, _RB)],
                        buf.at[p_static],
                        sem,
                    ).wait()

                issue(t + 2, (p_static + 2) % 3)

                start = blk_start(t)

                def scan_list(lv, carry):
                    ec, fl = carry
                    m = tl[pl.ds(lv * L, L)] == t
                    km = plsc.all_reduce_population_count(m)[0]

                    def matched(carry):
                        ec, fl = carry
                        lvv = idxl[pl.ds(lv * L, L)]
                        pv = plsc.cumsum(m.astype(jnp.int32)) - 1
                        plsc.store_scatter(jtmp, [pv], lvv, mask=m)
                        jlv = jl[pl.ds(lv * L, L)]
                        plsc.store_scatter(jtmp, [pv + L], jlv, mask=m)

                        def entry(e, carry):
                            ec, fl = carry
                            espl = jnp.full((L,), e, jnp.int32)
                            ev0 = plsc.load_gather(jtmp, [espl])[0]
                            jev = plsc.load_gather(jtmp, [espl + L])[0]
                            lv2 = jnp.full((L,), ev0 - start, jnp.int32)
                            pspl = jnp.full((L,), p_static, jnp.int32)
                            for kk in range(D // L):
                                ext[ec, pl.ds(kk * L, L)] = plsc.load_gather(
                                    buf, [pspl, lanes + kk * L, lv2]
                                )
                            return emit_append(jev, ec, fl)

                        return lax.fori_loop(0, km, entry, (ec, fl))

                    return lax.cond(km > 0, matched, lambda cc: cc, (ec, fl))

                if not _DIAG_NOSCAN:
                    carry = lax.fori_loop(0, nlv, scan_list, carry)
                return carry

            r3 = lax.rem(t, 3)
            return lax.cond(
                r3 == 0,
                lambda cc: with_parity(0, cc),
                lambda cc: lax.cond(
                    r3 == 1,
                    lambda c2: with_parity(1, c2),
                    lambda c2: with_parity(2, c2),
                    cc,
                ),
                carry,
            )

        if _BISECT_PHASE2:
            ec, fl = lax.fori_loop(0, tmax, block_body, (ec, fl))

        @pl.when(ec > 0)
        def _final_flush():
            pltpu.sync_copy(ext, rows_hbm.at[pl.ds(bw + fl * 64, 64)])
            pltpu.sync_copy(pbuf, pos_hbm.at[pl.ds(bw + fl * 64, 64)])

    return collect_kernel


@functools.cache
def _build_unscatter(B: int, D: int):
    info = plsc.get_sparse_core_info()
    nc = info.num_cores
    nw = nc * info.num_subcores
    spw = _NSLOT // nw
    npv = spw // 128

    mesh = plsc.VectorSubcoreMesh(core_axis_name="c", subcore_axis_name="s")

    @functools.partial(
        pl.kernel,
        mesh=mesh,
        out_type=jax.ShapeDtypeStruct((B, D), jnp.float32),
        scratch_types=[
            pltpu.VMEM((npv, 128), jnp.int32),
            pltpu.VMEM((spw, D), jnp.float32),
            pltpu.SemaphoreType.DMA,
        ],
        compiler_params=pltpu.CompilerParams(use_tc_tiling_on_sc=False),
    )
    def unscatter_kernel(rows_hbm, pos_hbm, out_hbm, pv, rv, sem):
        w = lax.axis_index("s") * nc + lax.axis_index("c")
        base = w * spw
        pltpu.sync_copy(pos_hbm.at[pl.ds(w * npv, npv)], pv)
        pltpu.sync_copy(rows_hbm.at[pl.ds(base, spw)], rv)
        copies = []
        for k in range(npv):
            copies.append(
                pltpu.async_copy(
                    rv.at[pl.ds(k * 128, 128)],
                    out_hbm.at[plsc.Indices(pv.at[k], ignored_value=-1)],
                    sem,
                )
            )
        for c in copies:
            c.wait()

    return unscatter_kernel


def kernel(data, emb):
    (B,) = data.shape
    V, D = emb.shape
    nw = 32
    tile_of = lax.bitwise_and(lax.shift_right_logical(data, 8), nw - 1)
    counts = jnp.zeros((nw,), jnp.int32).at[tile_of].add(1)
    regsz = ((counts + 127) // 128) * 128 + _SLACK
    base = jnp.concatenate(
        [jnp.zeros((1,), jnp.int32), jnp.cumsum(regsz, dtype=jnp.int32)]
    )
    base_pairs = jnp.pad(
        jnp.stack([base[:nw], base[1 : nw + 1]], axis=1), ((0, 0), (0, 14))
    )
    rows, pos = _build_collect(B, V, D)(data, emb.T, base_pairs)
    pos2 = pos.reshape(_NSLOT // 128, 128)
    return _build_unscatter(B, D)(rows, pos2)


# speedup vs baseline: 2.5682x; 1.0219x over previous
import functools

import jax
import jax.numpy as jnp
from jax import lax
from jax.experimental import pallas as pl
from jax.experimental.pallas import tpu as pltpu
from jax.experimental.pallas import tpu_sc as plsc

_RB = 256
_CAP = 1024
_SLACK = 0
_NSLOT = 20480
_SENTINEL = 0x7FFFFFF0
_BISECT_PHASE2 = True
_BISECT_OVERFLOW = True
_DIAG_NOSCAN = False


@functools.cache
def _build_collect(B: int, V: int, D: int):
    info = plsc.get_sparse_core_info()
    L = info.num_lanes
    nc = info.num_cores
    nw = nc * info.num_subcores
    nblk = (V + _RB - 1) // _RB
    tmax = (nblk + nw - 1) // nw
    n_chunks = 8
    chunk = B // n_chunks

    mesh = plsc.VectorSubcoreMesh(core_axis_name="c", subcore_axis_name="s")

    @functools.partial(
        pl.kernel,
        mesh=mesh,
        out_type=[
            jax.ShapeDtypeStruct((_NSLOT, D), jnp.float32),
            jax.ShapeDtypeStruct((_NSLOT,), jnp.int32),
        ],
        scratch_types=[
            pltpu.VMEM((chunk,), jnp.int32),
            pltpu.VMEM((_CAP + L,), jnp.int32),
            pltpu.VMEM((_CAP + L,), jnp.int32),
            pltpu.VMEM((_CAP + L,), jnp.int32),
            pltpu.VMEM((3, D, _RB), jnp.float32),
            pltpu.VMEM((64, D), jnp.float32),
            pltpu.VMEM((64,), jnp.int32),
            pltpu.VMEM((2 * L,), jnp.int32),
            pltpu.VMEM((D, 128), jnp.float32),
            pltpu.VMEM((nw, L), jnp.int32),
            pltpu.SemaphoreType.DMA,
            pltpu.SemaphoreType.DMA,
            pltpu.SemaphoreType.DMA,
        ],
        compiler_params=pltpu.CompilerParams(needs_layout_passes=False),
    )
    def collect_kernel(
        idx_hbm,
        table_hbm,
        base_hbm,
        rows_hbm,
        pos_hbm,
        dchunk,
        idxl,
        jl,
        tl,
        buf,
        ext,
        pbuf,
        jtmp,
        rtmp,
        basev,
        sem0,
        sem1,
        sem2,
    ):
        w = lax.axis_index("s") * nc + lax.axis_index("c")
        lanes = lax.iota(jnp.int32, L)
        neg1 = jnp.full((L,), -1, jnp.int32)

        pltpu.sync_copy(base_hbm, basev)
        bvec = basev[w, pl.ds(0, L)]
        bw = pl.multiple_of(bvec[0], 128)
        bw1 = pl.multiple_of(bvec[1], 128)

        def reset_pbuf():
            for i in range(64 // L):
                pbuf[pl.ds(i * L, L)] = neg1

        reset_pbuf()
        fill_end = jnp.where(w == nw - 1, jnp.int32(_NSLOT), bw1)
        nfill = (fill_end - bw) // 64

        def fill_body(f, carry):
            pltpu.sync_copy(pbuf, pos_hbm.at[pl.ds(bw + f * 64, 64)])
            return carry

        lax.fori_loop(0, nfill, fill_body, 0)

        def emit_append(jscalar, ec, fl):
            plsc.store_scatter(
                pbuf, [jnp.full((L,), ec, jnp.int32)],
                jnp.full((L,), jscalar, jnp.int32), mask=lanes == 0,
            )
            ec = ec + 1

            def flush(fl):
                pltpu.sync_copy(ext, rows_hbm.at[pl.ds(bw + fl * 64, 64)])
                pltpu.sync_copy(pbuf, pos_hbm.at[pl.ds(bw + fl * 64, 64)])
                reset_pbuf()
                return fl + 1

            fl = lax.cond(ec == 64, flush, lambda f: f, fl)
            ec = jnp.where(ec == 64, 0, ec)
            return ec, fl

        def scan_chunk(c, carry):
            pltpu.sync_copy(idx_hbm.at[pl.ds(c * chunk, chunk)], dchunk)

            def scan_vec(g, carry):
                cnt, ec, fl = carry
                v = dchunk[pl.ds(g * L, L)]
                mine = lax.bitwise_and(lax.shift_right_logical(v, 8), nw - 1) == w
                tailm = v >= (nblk - 1) * _RB
                listm = jnp.logical_and(mine, jnp.logical_not(tailm))
                k = plsc.all_reduce_population_count(listm)[0]
                jv = lanes + (c * chunk + g * L)
                in_cap = cnt + k <= _CAP

                @pl.when(jnp.logical_and(k > 0, in_cap))
                def _listed():
                    pv = cnt + plsc.cumsum(listm.astype(jnp.int32)) - 1
                    plsc.store_scatter(idxl, [pv], v, mask=listm)
                    plsc.store_scatter(jl, [pv], jv, mask=listm)

                ovm = jnp.where(in_cap, jnp.logical_and(mine, tailm), mine)
                novm = plsc.all_reduce_population_count(ovm)[0]

                def _overflow(carry):
                    ec, fl = carry
                    mine_i32 = ovm.astype(jnp.int32)
                    for i in range(L):
                        def do_ov(carry, i=i):
                            ec, fl = carry
                            col = pl.multiple_of(
                                lax.bitwise_and(v[i], ~jnp.int32(127)), 128
                            )
                            pltpu.sync_copy(
                                table_hbm.at[pl.ds(0, D), pl.ds(col, 128)], rtmp
                            )
                            l2 = lax.bitwise_and(v[i], 127)
                            l2v = jnp.full((L,), l2, jnp.int32)
                            for kk in range(D // L):
                                ext[ec, pl.ds(kk * L, L)] = plsc.load_gather(
                                    rtmp, [lanes + kk * L, l2v]
                                )
                            return emit_append(jv[i], ec, fl)

                        carry = lax.cond(mine_i32[i] != 0, do_ov, lambda cc: cc, (ec, fl))
                        ec, fl = carry
                    return ec, fl

                ec, fl = lax.cond(novm > 0, _overflow, lambda cc: cc, (ec, fl))
                cnt = jnp.where(in_cap, cnt + k, cnt)
                return cnt, ec, fl

            return lax.fori_loop(0, chunk // L, scan_vec, carry)

        lc, ec, fl = lax.fori_loop(
            0, n_chunks, scan_chunk, (jnp.int32(0), jnp.int32(0), jnp.int32(0))
        )

        plsc.store_scatter(
            idxl, [lc + lanes], jnp.full((L,), _SENTINEL, jnp.int32)
        )
        nlv = (lc + L - 1) // L

        def tconv(i, carry):
            tl[pl.ds(i * L, L)] = lax.shift_right_logical(idxl[pl.ds(i * L, L)], 13)
            return carry

        lax.fori_loop(0, nlv + 1, tconv, 0)

        nfull = nblk - 1

        def blk_start(t):
            return pl.multiple_of((w + nw * t) * _RB, 128)

        sems = (sem0, sem1, sem2)

        def issue(t, p_static):
            sem = sems[p_static]

            @pl.when(w + nw * t < nfull)
            def _():
                for gg in range(D // 8):
                    pltpu.async_copy(
                        table_hbm.at[pl.ds(gg * 8, 8), pl.ds(blk_start(t), _RB)],
                        buf.at[p_static, pl.ds(gg * 8, 8)],
                        sem,
                    )

        issue(jnp.int32(0), 0)
        issue(jnp.int32(1), 1)

        def block_body(t, carry):
            def with_parity(p_static, carry):
                sem = sems[p_static]

                @pl.when(w + nw * t < nfull)
                def _wait():
                    pltpu.make_async_copy(
                        table_hbm.at[pl.ds(0, D), pl.ds(0, _RB)],
                        buf.at[p_static],
                        sem,
                    ).wait()

                issue(t + 2, (p_static + 2) % 3)

                start = blk_start(t)

                def scan_list(lv, carry):
                    ec, fl = carry
                    m = tl[pl.ds(lv * L, L)] == t
                    km = plsc.all_reduce_population_count(m)[0]

                    def matched(carry):
                        ec, fl = carry
                        lvv = idxl[pl.ds(lv * L, L)]
                        pv = plsc.cumsum(m.astype(jnp.int32)) - 1
                        plsc.store_scatter(jtmp, [pv], lvv, mask=m)
                        jlv = jl[pl.ds(lv * L, L)]
                        plsc.store_scatter(jtmp, [pv + L], jlv, mask=m)

                        def entry(e, carry):
                            ec, fl = carry
                            espl = jnp.full((L,), e, jnp.int32)
                            ev0 = plsc.load_gather(jtmp, [espl])[0]
                            jev = plsc.load_gather(jtmp, [espl + L])[0]
                            lv2 = jnp.full((L,), ev0 - start, jnp.int32)
                            pspl = jnp.full((L,), p_static, jnp.int32)
                            for kk in range(D // L):
                                ext[ec, pl.ds(kk * L, L)] = plsc.load_gather(
                                    buf, [pspl, lanes + kk * L, lv2]
                                )
                            return emit_append(jev, ec, fl)

                        return lax.fori_loop(0, km, entry, (ec, fl))

                    return lax.cond(km > 0, matched, lambda cc: cc, (ec, fl))

                if not _DIAG_NOSCAN:
                    carry = lax.fori_loop(0, nlv, scan_list, carry)
                return carry

            r3 = lax.rem(t, 3)
            return lax.cond(
                r3 == 0,
                lambda cc: with_parity(0, cc),
                lambda cc: lax.cond(
                    r3 == 1,
                    lambda c2: with_parity(1, c2),
                    lambda c2: with_parity(2, c2),
                    cc,
                ),
                carry,
            )

        if _BISECT_PHASE2:
            ec, fl = lax.fori_loop(0, tmax, block_body, (ec, fl))

        @pl.when(ec > 0)
        def _final_flush():
            pltpu.sync_copy(ext, rows_hbm.at[pl.ds(bw + fl * 64, 64)])
            pltpu.sync_copy(pbuf, pos_hbm.at[pl.ds(bw + fl * 64, 64)])

    return collect_kernel


@functools.cache
def _build_unscatter(B: int, D: int):
    info = plsc.get_sparse_core_info()
    nc = info.num_cores
    nw = nc * info.num_subcores
    spw = _NSLOT // nw
    npv = spw // 128

    mesh = plsc.VectorSubcoreMesh(core_axis_name="c", subcore_axis_name="s")

    @functools.partial(
        pl.kernel,
        mesh=mesh,
        out_type=jax.ShapeDtypeStruct((B, D), jnp.float32),
        scratch_types=[
            pltpu.VMEM((npv, 128), jnp.int32),
            pltpu.VMEM((spw, D), jnp.float32),
            pltpu.SemaphoreType.DMA,
        ],
        compiler_params=pltpu.CompilerParams(use_tc_tiling_on_sc=False),
    )
    def unscatter_kernel(rows_hbm, pos_hbm, out_hbm, pv, rv, sem):
        w = lax.axis_index("s") * nc + lax.axis_index("c")
        base = w * spw
        pltpu.sync_copy(pos_hbm.at[pl.ds(w * npv, npv)], pv)
        pltpu.sync_copy(rows_hbm.at[pl.ds(base, spw)], rv)
        copies = []
        for k in range(npv):
            copies.append(
                pltpu.async_copy(
                    rv.at[pl.ds(k * 128, 128)],
                    out_hbm.at[plsc.Indices(pv.at[k], ignored_value=-1)],
                    sem,
                )
            )
        for c in copies:
            c.wait()

    return unscatter_kernel


def kernel(data, emb):
    (B,) = data.shape
    V, D = emb.shape
    nw = 32
    tile_of = lax.bitwise_and(lax.shift_right_logical(data, 8), nw - 1)
    counts = jnp.sum(
        (tile_of[:, None] == jnp.arange(nw, dtype=jnp.int32)[None, :]).astype(
            jnp.int32
        ),
        axis=0,
    )
    regsz = ((counts + 127) // 128) * 128 + _SLACK
    base = jnp.concatenate(
        [jnp.zeros((1,), jnp.int32), jnp.cumsum(regsz, dtype=jnp.int32)]
    )
    base_pairs = jnp.pad(
        jnp.stack([base[:nw], base[1 : nw + 1]], axis=1), ((0, 0), (0, 14))
    )
    rows, pos = _build_collect(B, V, D)(data, emb.T, base_pairs)
    pos2 = pos.reshape(_NSLOT // 128, 128)
    return _build_unscatter(B, D)(rows, pos2)
